# Initial kernel scaffold; baseline (speedup 1.0000x reference)
import jax
import jax.numpy as jnp
from jax.experimental import pallas as pl


def kernel(input):
    raise NotImplementedError("write your pallas kernel here")


# speedup vs baseline: 10.0539x; 10.0539x over previous
import functools

import jax
import jax.numpy as jnp
from jax import lax
from jax.experimental import pallas as pl
from jax.experimental.pallas import tpu as pltpu
from jax.experimental.pallas import tpu_sc as plsc

_B = 32
_N = 96 * 56 * 56
_KK = int(0.1 * _N)
_CHUNK = 37632
_NCHUNK = _N // _CHUNK
_NC = 2
_NS = 16
_LEVELS = ((20, 11), (10, 10), (0, 10))
_MAXBINS = 1 << 11

_mesh = plsc.VectorSubcoreMesh(core_axis_name="c", subcore_axis_name="s")


@functools.partial(
    pl.kernel,
    out_type=jax.ShapeDtypeStruct((_B, 16), jnp.int32),
    mesh=_mesh,
    compiler_params=pltpu.CompilerParams(needs_layout_passes=False),
    scratch_types=[
        pltpu.VMEM((_CHUNK,), jnp.int32),
        pltpu.VMEM((_MAXBINS * 16,), jnp.int32),
        pltpu.VMEM((16,), jnp.int32),
    ],
)
def _sc_row_thresholds(x_hbm, out_hbm, buf, histp, thrbuf):
    lane = lax.iota(jnp.int32, 16)
    ones = jnp.ones((16,), jnp.int32)
    row = lax.axis_index("s") * _NC + lax.axis_index("c")

    rank = jnp.int32(_KK)
    prefix = jnp.int32(0)
    minvec = jnp.full((16,), jnp.int32(0x7FFFFFFF), jnp.int32)

    for level, (shift, bits) in enumerate(_LEVELS):
        nbins = 1 << bits
        pshift = shift + bits
        track_min = level == 0

        def zero_body(i, carry):
            histp[pl.ds(i * 16, 16)] = jnp.zeros((16,), jnp.int32)
            return carry

        lax.fori_loop(0, nbins, zero_body, jnp.int32(0))

        pvec = jnp.full((16,), prefix, jnp.int32)
        binmask = jnp.int32(nbins - 1)

        def vec_body(i, mn, pshift=pshift, shift=shift, pvec=pvec,
                     binmask=binmask, track_min=track_min):
            ki = buf[pl.ds(i * 16, 16)]
            k = ki & jnp.int32(0x7FFFFFFF)
            match = (k >> pshift) == pvec
            bn = (k >> shift) & binmask
            plsc.addupdate_scatter(histp, [(bn << 4) + lane], ones, mask=match)
            if track_min:
                m = ki ^ ((ki >> 31) & jnp.int32(0x7FFFFFFF))
                mn = jnp.minimum(mn, m)
            return mn

        def chunk_body(c, mn, vec_body=vec_body):
            pltpu.sync_copy(x_hbm.at[row, pl.ds(c * _CHUNK, _CHUNK)], buf)
            return lax.fori_loop(0, _CHUNK // 16, vec_body, mn)

        minvec = lax.fori_loop(0, _NCHUNK, chunk_body, minvec)

        def scan_body(j, carry, nbins=nbins, rank=rank):
            bstar, below, cum = carry
            b = jnp.int32(nbins - 1) - j
            cnt = jnp.sum(histp[pl.ds(b * 16, 16)])
            newcum = cum + cnt
            hit = jnp.logical_and(newcum >= rank, bstar < 0)
            bstar = jnp.where(hit, b, bstar)
            below = jnp.where(hit, cum, below)
            return bstar, below, newcum

        bstar, above_cnt, _ = lax.fori_loop(
            0, nbins, scan_body,
            (jnp.int32(-1), jnp.int32(0), jnp.int32(0)))
        rank = rank - above_cnt
        prefix = (prefix << bits) | bstar

    m_min = jnp.min(minvec)
    cond = m_min <= ~prefix
    row_min_bits = m_min ^ ((m_min >> 31) & jnp.int32(0x7FFFFFFF))
    thr_bits = jnp.where(cond, row_min_bits, prefix)
    thrbuf[...] = jnp.full((16,), thr_bits, jnp.int32)
    pltpu.sync_copy(thrbuf, out_hbm.at[row])


def _mask_body(x_ref, t_ref, o_ref):
    x = x_ref[...]
    o_ref[...] = jnp.where(x >= t_ref[:, :1], x, jnp.float32(0.0))


_mask = pl.pallas_call(
    _mask_body,
    grid=(_B // 8, _NCHUNK),
    in_specs=[
        pl.BlockSpec((8, _CHUNK), lambda i, j: (i, j)),
        pl.BlockSpec((8, 16), lambda i, j: (i, 0)),
    ],
    out_specs=pl.BlockSpec((8, _CHUNK), lambda i, j: (i, j)),
    out_shape=jax.ShapeDtypeStruct((_B, _N), jnp.float32),
)


def kernel(input):
    flat = input.reshape(_B, _N)
    bits = lax.bitcast_convert_type(flat, jnp.int32)
    thr_bits = _sc_row_thresholds(bits)
    thr16 = lax.bitcast_convert_type(thr_bits, jnp.float32)
    out = _mask(flat, thr16)
    return out.reshape(input.shape)


# speedup vs baseline: 11.2357x; 1.1175x over previous
import functools

import jax
import jax.numpy as jnp
from jax import lax
from jax.experimental import pallas as pl
from jax.experimental.pallas import tpu as pltpu
from jax.experimental.pallas import tpu_sc as plsc

_B = 32
_N = 96 * 56 * 56
_KK = int(0.1 * _N)
_CHUNK = 37632
_NCHUNK = _N // _CHUNK
_NC = 2
_NS = 16
_UNROLL = 4
_MAXBINS = 1 << 11

_mesh = plsc.VectorSubcoreMesh(core_axis_name="c", subcore_axis_name="s")


@functools.partial(
    pl.kernel,
    out_type=jax.ShapeDtypeStruct((_B, 16), jnp.float32),
    mesh=_mesh,
    compiler_params=pltpu.CompilerParams(needs_layout_passes=False),
    scratch_types=[
        pltpu.VMEM((_CHUNK,), jnp.float32),
        pltpu.VMEM((_CHUNK,), jnp.float32),
        pltpu.VMEM((_MAXBINS * 16,), jnp.int32),
        pltpu.VMEM((16,), jnp.float32),
        pltpu.SemaphoreType.DMA,
    ],
)
def _sc_row_thresholds(x_hbm, out_hbm, buf0, buf1, histp, thrbuf, sem):
    lane = lax.iota(jnp.int32, 16)
    ones = jnp.ones((16,), jnp.int32)
    row = lax.axis_index("s") * _NC + lax.axis_index("c")
    bufs = (buf0, buf1)

    rank = jnp.int32(_KK)
    prefix = jnp.int32(0)
    minvec = jnp.full((16,), jnp.inf, jnp.float32)

    for level, (shift, bits) in enumerate(((20, 11), (10, 10), (0, 10))):
        nbins = 1 << bits
        pshift = shift + bits
        pmask = jnp.int32((1 << (31 - pshift)) - 1)
        track_min = level == 0

        def zero_body(i, carry):
            histp[pl.ds(i * 64, 16)] = jnp.zeros((16,), jnp.int32)
            histp[pl.ds(i * 64 + 16, 16)] = jnp.zeros((16,), jnp.int32)
            histp[pl.ds(i * 64 + 32, 16)] = jnp.zeros((16,), jnp.int32)
            histp[pl.ds(i * 64 + 48, 16)] = jnp.zeros((16,), jnp.int32)
            return carry

        lax.fori_loop(0, nbins * 16 // 64, zero_body, jnp.int32(0))

        pvec = jnp.full((16,), prefix, jnp.int32)
        idxmask = jnp.int32((nbins - 1) << 4)

        def slot(v, mn, pshift=pshift, shift=shift, pvec=pvec,
                 idxmask=idxmask, pmask=pmask, track_min=track_min,
                 level=level):
            ki = plsc.bitcast(v, jnp.int32)
            idx = ((ki >> (shift - 4)) & idxmask if shift >= 4
                   else (ki << (4 - shift)) & idxmask) + lane
            if level == 0:
                plsc.addupdate_scatter(histp, [idx], ones)
            else:
                match = ((ki >> pshift) & pmask) == pvec
                plsc.addupdate_scatter(histp, [idx], ones, mask=match)
            if track_min:
                mn = jnp.minimum(mn, v)
            return mn

        def vec_body(i, mn, buf=None, slot=slot):
            base = i * (16 * _UNROLL)
            for u in range(_UNROLL):
                mn = slot(buf[pl.ds(base + u * 16, 16)], mn)
            return mn

        h = pltpu.async_copy(x_hbm.at[row, pl.ds(0, _CHUNK)], bufs[0], sem)
        for c in range(_NCHUNK):
            h.wait()
            if c + 1 < _NCHUNK:
                h = pltpu.async_copy(
                    x_hbm.at[row, pl.ds((c + 1) * _CHUNK, _CHUNK)],
                    bufs[(c + 1) % 2], sem)
            minvec = lax.fori_loop(
                0, _CHUNK // (16 * _UNROLL),
                functools.partial(vec_body, buf=bufs[c % 2]), minvec)

        def scan_body(j, carry, nbins=nbins, rank=rank):
            bstar, below, cum = carry
            b = jnp.int32(nbins - 1) - j
            cnt = jnp.sum(histp[pl.ds(b * 16, 16)])
            newcum = cum + cnt
            hit = jnp.logical_and(newcum >= rank, bstar < 0)
            bstar = jnp.where(hit, b, bstar)
            below = jnp.where(hit, cum, below)
            return bstar, below, newcum

        bstar, above_cnt, _ = lax.fori_loop(
            0, nbins, scan_body,
            (jnp.int32(-1), jnp.int32(0), jnp.int32(0)))
        rank = rank - above_cnt
        prefix = (prefix << bits) | bstar

    t = jnp.max(plsc.bitcast(jnp.full((16,), prefix, jnp.int32), jnp.float32))
    row_min = jnp.min(minvec)
    thr = jnp.where(row_min <= -t, row_min, t)
    thrbuf[...] = jnp.full((16,), thr, jnp.float32)
    pltpu.sync_copy(thrbuf, out_hbm.at[row])


def _mask_body(x_ref, t_ref, o_ref):
    x = x_ref[...]
    o_ref[...] = jnp.where(x >= t_ref[:, :1], x, jnp.float32(0.0))


_mask = pl.pallas_call(
    _mask_body,
    grid=(_B // 8, _NCHUNK),
    in_specs=[
        pl.BlockSpec((8, _CHUNK), lambda i, j: (i, j)),
        pl.BlockSpec((8, 16), lambda i, j: (i, 0)),
    ],
    out_specs=pl.BlockSpec((8, _CHUNK), lambda i, j: (i, j)),
    out_shape=jax.ShapeDtypeStruct((_B, _N), jnp.float32),
)


def kernel(input):
    flat = input.reshape(_B, _N)
    thr16 = _sc_row_thresholds(flat)
    out = _mask(flat, thr16)
    return out.reshape(input.shape)


# speedup vs baseline: 11.9986x; 1.0679x over previous
import functools

import jax
import jax.numpy as jnp
from jax import lax
from jax.experimental import pallas as pl
from jax.experimental.pallas import tpu as pltpu
from jax.experimental.pallas import tpu_sc as plsc

_B = 32
_N = 96 * 56 * 56
_KK = int(0.1 * _N)
_CHUNK = 37632
_NCHUNK = _N // _CHUNK
_NC = 2
_NS = 16
_UNROLL = 8
_MAXBINS = 1 << 11

_mesh = plsc.VectorSubcoreMesh(core_axis_name="c", subcore_axis_name="s")


@functools.partial(
    pl.kernel,
    out_type=jax.ShapeDtypeStruct((_B, 128), jnp.float32),
    mesh=_mesh,
    compiler_params=pltpu.CompilerParams(needs_layout_passes=False),
    scratch_types=[
        pltpu.VMEM((_CHUNK,), jnp.float32),
        pltpu.VMEM((_CHUNK,), jnp.float32),
        pltpu.VMEM((_MAXBINS * 16,), jnp.int32),
        pltpu.VMEM((128,), jnp.float32),
        pltpu.SemaphoreType.DMA,
    ],
)
def _sc_row_thresholds(x_hbm, out_hbm, buf0, buf1, histp, thrbuf, sem):
    lane = lax.iota(jnp.int32, 16)
    ones = jnp.ones((16,), jnp.int32)
    row = lax.axis_index("s") * _NC + lax.axis_index("c")
    bufs = (buf0, buf1)

    rank = jnp.int32(_KK)
    prefix = jnp.int32(0)
    minvec = jnp.full((16,), jnp.inf, jnp.float32)

    for level, (shift, bits) in enumerate(((20, 11), (10, 10), (0, 10))):
        nbins = 1 << bits
        pshift = shift + bits
        pmask = jnp.int32((1 << (31 - pshift)) - 1)
        track_min = level == 0

        def zero_body(i, carry):
            histp[pl.ds(i * 64, 16)] = jnp.zeros((16,), jnp.int32)
            histp[pl.ds(i * 64 + 16, 16)] = jnp.zeros((16,), jnp.int32)
            histp[pl.ds(i * 64 + 32, 16)] = jnp.zeros((16,), jnp.int32)
            histp[pl.ds(i * 64 + 48, 16)] = jnp.zeros((16,), jnp.int32)
            return carry

        lax.fori_loop(0, nbins * 16 // 64, zero_body, jnp.int32(0))

        pvec = jnp.full((16,), prefix, jnp.int32)
        idxmask = jnp.int32((nbins - 1) << 4)

        def slot(v, mn, pshift=pshift, shift=shift, pvec=pvec,
                 idxmask=idxmask, pmask=pmask, track_min=track_min,
                 level=level):
            ki = plsc.bitcast(v, jnp.int32)
            idx = ((ki >> (shift - 4)) & idxmask if shift >= 4
                   else (ki << (4 - shift)) & idxmask) + lane
            if level == 0:
                plsc.addupdate_scatter(histp, [idx], ones)
            else:
                match = ((ki >> pshift) & pmask) == pvec
                plsc.addupdate_scatter(histp, [idx], ones, mask=match)
            if track_min:
                mn = jnp.minimum(mn, v)
            return mn

        def vec_body(i, mn, buf=None, slot=slot):
            base = i * (16 * _UNROLL)
            for u in range(_UNROLL):
                mn = slot(buf[pl.ds(base + u * 16, 16)], mn)
            return mn

        rowbase = row * _N
        h = pltpu.async_copy(x_hbm.at[pl.ds(rowbase, _CHUNK)], bufs[0], sem)
        for c in range(_NCHUNK):
            h.wait()
            if c + 1 < _NCHUNK:
                h = pltpu.async_copy(
                    x_hbm.at[pl.ds(rowbase + (c + 1) * _CHUNK, _CHUNK)],
                    bufs[(c + 1) % 2], sem)
            minvec = lax.fori_loop(
                0, _CHUNK // (16 * _UNROLL),
                functools.partial(vec_body, buf=bufs[c % 2]), minvec)

        def scan_body(j, carry, nbins=nbins, rank=rank):
            bstar, below, cum = carry
            b = jnp.int32(nbins - 1) - j
            cnt = jnp.sum(histp[pl.ds(b * 16, 16)])
            newcum = cum + cnt
            hit = jnp.logical_and(newcum >= rank, bstar < 0)
            bstar = jnp.where(hit, b, bstar)
            below = jnp.where(hit, cum, below)
            return bstar, below, newcum

        bstar, above_cnt, _ = lax.fori_loop(
            0, nbins, scan_body,
            (jnp.int32(-1), jnp.int32(0), jnp.int32(0)))
        rank = rank - above_cnt
        prefix = (prefix << bits) | bstar

    t = jnp.max(plsc.bitcast(jnp.full((16,), prefix, jnp.int32), jnp.float32))
    row_min = jnp.min(minvec)
    thr = jnp.where(row_min <= -t, row_min, t)
    thrv = jnp.full((16,), thr, jnp.float32)
    for u in range(8):
        thrbuf[pl.ds(u * 16, 16)] = thrv
    pltpu.sync_copy(thrbuf, out_hbm.at[row])


def _mask_body(x_ref, t_ref, o_ref):
    x = x_ref[...]
    o_ref[...] = jnp.where(x >= t_ref[:, :1], x, jnp.float32(0.0))


_mask = pl.pallas_call(
    _mask_body,
    grid=(_B // 8, _NCHUNK),
    in_specs=[
        pl.BlockSpec((8, _CHUNK), lambda i, j: (i, j)),
        pl.BlockSpec((8, 128), lambda i, j: (i, 0)),
    ],
    out_specs=pl.BlockSpec((8, _CHUNK), lambda i, j: (i, j)),
    out_shape=jax.ShapeDtypeStruct((_B, _N), jnp.float32),
)


def kernel(input):
    flat = input.reshape(_B, _N)
    thr = _sc_row_thresholds(input.reshape(_B * _N))
    out = _mask(flat, thr)
    return out.reshape(input.shape)


# speedup vs baseline: 19.5211x; 1.6270x over previous
import functools

import jax
import jax.numpy as jnp
from jax import lax
from jax.experimental import pallas as pl
from jax.experimental.pallas import tpu as pltpu
from jax.experimental.pallas import tpu_sc as plsc

_B = 32
_N = 96 * 56 * 56
_KK = int(0.1 * _N)
_CHUNK = 37632
_NCHUNK = _N // _CHUNK
_NC = 2
_NS = 16
_UNROLL = 8
_MAXBINS = 1 << 11

_mesh = plsc.VectorSubcoreMesh(core_axis_name="c", subcore_axis_name="s")


@functools.partial(
    pl.kernel,
    out_type=jax.ShapeDtypeStruct((_B, 128), jnp.float32),
    mesh=_mesh,
    compiler_params=pltpu.CompilerParams(needs_layout_passes=False),
    scratch_types=[
        pltpu.VMEM((_CHUNK,), jnp.float32),
        pltpu.VMEM((_CHUNK,), jnp.float32),
        pltpu.VMEM((_MAXBINS * 16,), jnp.int32),
        pltpu.VMEM((128,), jnp.float32),
        pltpu.SemaphoreType.DMA,
    ],
)
def _sc_row_thresholds(x_hbm, out_hbm, buf0, buf1, histp, thrbuf, sem):
    lane = lax.iota(jnp.int32, 16)
    ones = jnp.ones((16,), jnp.int32)
    row = lax.axis_index("s") * _NC + lax.axis_index("c")
    bufs = (buf0, buf1)

    rank = jnp.int32(_KK)
    prefix = jnp.int32(0)
    minvec = jnp.full((16,), jnp.inf, jnp.float32)

    for level, (shift, bits) in enumerate(((20, 11), (10, 10), (0, 10))):
        nbins = 1 << bits
        pshift = shift + bits
        pmask = jnp.int32((1 << (31 - pshift)) - 1)
        track_min = level == 0

        def zero_body(i, carry):
            histp[pl.ds(i * 64, 16)] = jnp.zeros((16,), jnp.int32)
            histp[pl.ds(i * 64 + 16, 16)] = jnp.zeros((16,), jnp.int32)
            histp[pl.ds(i * 64 + 32, 16)] = jnp.zeros((16,), jnp.int32)
            histp[pl.ds(i * 64 + 48, 16)] = jnp.zeros((16,), jnp.int32)
            return carry

        lax.fori_loop(0, nbins * 16 // 64, zero_body, jnp.int32(0))

        pvec = jnp.full((16,), prefix, jnp.int32)
        idxmask = jnp.int32((nbins - 1) << 4)

        def make_body(buf, pshift=pshift, shift=shift, pvec=pvec,
                      idxmask=idxmask, pmask=pmask, track_min=track_min,
                      level=level):
            def body(i, mn):
                v = buf[pl.ds(i * 16, 16)]
                ki = plsc.bitcast(v, jnp.int32)
                idx = ((ki >> (shift - 4)) & idxmask if shift >= 4
                       else (ki << (4 - shift)) & idxmask) + lane
                if level == 0:
                    plsc.addupdate_scatter(histp, [idx], ones)
                else:
                    match = ((ki >> pshift) & pmask) == pvec
                    plsc.addupdate_scatter(histp, [idx], ones, mask=match)
                if track_min:
                    mn = jnp.minimum(mn, v)
                return mn
            return body

        rowbase = row * _N
        h = pltpu.async_copy(x_hbm.at[pl.ds(rowbase, _CHUNK)], bufs[0], sem)
        for c in range(_NCHUNK):
            h.wait()
            if c + 1 < _NCHUNK:
                h = pltpu.async_copy(
                    x_hbm.at[pl.ds(rowbase + (c + 1) * _CHUNK, _CHUNK)],
                    bufs[(c + 1) % 2], sem)
            minvec = plsc.parallel_loop(
                0, _CHUNK // 16, 1, unroll=_UNROLL, carry=minvec,
            )(make_body(bufs[c % 2]))

        def scan_body(j, carry, nbins=nbins, rank=rank):
            bstar, below, cum = carry
            b = jnp.int32(nbins - 1) - j
            cnt = jnp.sum(histp[pl.ds(b * 16, 16)])
            newcum = cum + cnt
            hit = jnp.logical_and(newcum >= rank, bstar < 0)
            bstar = jnp.where(hit, b, bstar)
            below = jnp.where(hit, cum, below)
            return bstar, below, newcum

        bstar, above_cnt, _ = lax.fori_loop(
            0, nbins, scan_body,
            (jnp.int32(-1), jnp.int32(0), jnp.int32(0)))
        rank = rank - above_cnt
        prefix = (prefix << bits) | bstar

    t = jnp.max(plsc.bitcast(jnp.full((16,), prefix, jnp.int32), jnp.float32))
    row_min = jnp.min(minvec)
    thr = jnp.where(row_min <= -t, row_min, t)
    thrv = jnp.full((16,), thr, jnp.float32)
    for u in range(8):
        thrbuf[pl.ds(u * 16, 16)] = thrv
    pltpu.sync_copy(thrbuf, out_hbm.at[row])


def _mask_body(x_ref, t_ref, o_ref):
    x = x_ref[...]
    o_ref[...] = jnp.where(x >= t_ref[:, :1], x, jnp.float32(0.0))


_mask = pl.pallas_call(
    _mask_body,
    grid=(_B // 8, _NCHUNK),
    in_specs=[
        pl.BlockSpec((8, _CHUNK), lambda i, j: (i, j)),
        pl.BlockSpec((8, 128), lambda i, j: (i, 0)),
    ],
    out_specs=pl.BlockSpec((8, _CHUNK), lambda i, j: (i, j)),
    out_shape=jax.ShapeDtypeStruct((_B, _N), jnp.float32),
)


def kernel(input):
    flat = input.reshape(_B, _N)
    thr = _sc_row_thresholds(input.reshape(_B * _N))
    out = _mask(flat, thr)
    return out.reshape(input.shape)


# speedup vs baseline: 26.8184x; 1.3738x over previous
import functools

import jax
import jax.numpy as jnp
from jax import lax
from jax.experimental import pallas as pl
from jax.experimental.pallas import tpu as pltpu
from jax.experimental.pallas import tpu_sc as plsc

_B = 32
_N = 96 * 56 * 56
_KK = int(0.1 * _N)
_CHUNK = 37632
_NCHUNK = _N // _CHUNK
_NC = 2
_NS = 16
_UNROLL = 8
_MAXBINS = 1 << 11

_mesh = plsc.VectorSubcoreMesh(core_axis_name="c", subcore_axis_name="s")


@functools.partial(
    pl.kernel,
    out_type=jax.ShapeDtypeStruct((_B, 128), jnp.float32),
    mesh=_mesh,
    compiler_params=pltpu.CompilerParams(needs_layout_passes=False),
    scratch_types=[
        pltpu.VMEM((_CHUNK,), jnp.float32),
        pltpu.VMEM((_CHUNK,), jnp.float32),
        pltpu.VMEM((_MAXBINS * 16,), jnp.int32),
        pltpu.VMEM((128,), jnp.float32),
        pltpu.SemaphoreType.DMA,
    ],
)
def _sc_row_thresholds(x_hbm, out_hbm, buf0, buf1, histp, thrbuf, sem):
    lane = lax.iota(jnp.int32, 16)
    ones = jnp.ones((16,), jnp.int32)
    row = lax.axis_index("s") * _NC + lax.axis_index("c")
    bufs = (buf0, buf1)

    rank = jnp.int32(_KK)
    prefix = jnp.int32(0)
    minvec = jnp.full((16,), jnp.inf, jnp.float32)

    for level, (shift, bits) in enumerate(((20, 11), (10, 10), (0, 10))):
        nbins = 1 << bits
        pshift = shift + bits
        pmask = jnp.int32((1 << (31 - pshift)) - 1)
        track_min = level == 0

        def zero_body(i, carry):
            histp[pl.ds(i * 64, 16)] = jnp.zeros((16,), jnp.int32)
            histp[pl.ds(i * 64 + 16, 16)] = jnp.zeros((16,), jnp.int32)
            histp[pl.ds(i * 64 + 32, 16)] = jnp.zeros((16,), jnp.int32)
            histp[pl.ds(i * 64 + 48, 16)] = jnp.zeros((16,), jnp.int32)
            return carry

        lax.fori_loop(0, nbins * 16 // 64, zero_body, jnp.int32(0))

        pvec = jnp.full((16,), prefix, jnp.int32)
        idxmask = jnp.int32((nbins - 1) << 4)

        def make_body(buf, pshift=pshift, shift=shift, pvec=pvec,
                      idxmask=idxmask, pmask=pmask, track_min=track_min,
                      level=level):
            def body(i, mn):
                v = buf[pl.ds(i * 16, 16)]
                ki = plsc.bitcast(v, jnp.int32)
                idx = ((ki >> (shift - 4)) & idxmask if shift >= 4
                       else (ki << (4 - shift)) & idxmask) + lane
                if level == 0:
                    plsc.addupdate_scatter(histp, [idx], ones)
                else:
                    match = ((ki >> pshift) & pmask) == pvec
                    plsc.addupdate_scatter(histp, [idx], ones, mask=match)
                if track_min:
                    mn = jnp.minimum(mn, v)
                return mn
            return body

        rowbase = row * _N
        h = pltpu.async_copy(x_hbm.at[pl.ds(rowbase, _CHUNK)], bufs[0], sem)
        for c in range(_NCHUNK):
            h.wait()
            if c + 1 < _NCHUNK:
                h = pltpu.async_copy(
                    x_hbm.at[pl.ds(rowbase + (c + 1) * _CHUNK, _CHUNK)],
                    bufs[(c + 1) % 2], sem)
            minvec = plsc.parallel_loop(
                0, _CHUNK // 16, 1, unroll=_UNROLL, carry=minvec,
            )(make_body(bufs[c % 2]))

        def scan_body(j, carry, nbins=nbins, rank=rank):
            bstar, below, cum = carry
            b = jnp.int32(nbins - 1) - j
            cnt = jnp.sum(histp[pl.ds(b * 16, 16)])
            newcum = cum + cnt
            hit = jnp.logical_and(newcum >= rank, bstar < 0)
            bstar = jnp.where(hit, b, bstar)
            below = jnp.where(hit, cum, below)
            return bstar, below, newcum

        bstar, above_cnt, _ = lax.fori_loop(
            0, nbins, scan_body,
            (jnp.int32(-1), jnp.int32(0), jnp.int32(0)))
        rank = rank - above_cnt
        prefix = (prefix << bits) | bstar

    t = jnp.max(plsc.bitcast(jnp.full((16,), prefix, jnp.int32), jnp.float32))
    row_min = jnp.min(minvec)
    thr = jnp.where(row_min <= -t, row_min, t)
    thrv = jnp.full((16,), thr, jnp.float32)
    for u in range(8):
        thrbuf[pl.ds(u * 16, 16)] = thrv
    pltpu.sync_copy(thrbuf, out_hbm.at[row])


def _mask_body(x_ref, t_ref, o_ref):
    x = x_ref[...]
    o_ref[...] = jnp.where(x >= t_ref[0, 0, 0], x, jnp.float32(0.0))


_mask = pl.pallas_call(
    _mask_body,
    grid=(_B,),
    in_specs=[
        pl.BlockSpec((1, 96, 56, 56), lambda i: (i, 0, 0, 0)),
        pl.BlockSpec((1, 1, 128), lambda i: (i, 0, 0)),
    ],
    out_specs=pl.BlockSpec((1, 96, 56, 56), lambda i: (i, 0, 0, 0)),
    out_shape=jax.ShapeDtypeStruct((_B, 96, 56, 56), jnp.float32),
)


def kernel(input):
    thr = _sc_row_thresholds(input.reshape(_B * _N))
    return _mask(input, thr.reshape(_B, 1, 128))
